# Initial kernel scaffold; baseline (speedup 1.0000x reference)
import jax
import jax.numpy as jnp
from jax.experimental import pallas as pl


def kernel(idx, emb, W, b):
    raise NotImplementedError("write your pallas kernel here")


# speedup vs baseline: 2.9192x; 2.9192x over previous
import jax
import jax.numpy as jnp
from jax import lax
from jax.experimental import pallas as pl

_N = 2000
_K = 30
_D = 64
_ALPHA = 3.0
_BM = 200


def _embed_body(x_ref, w_ref, b_ref, n_ref):
    x = x_ref[:]
    w = w_ref[:]
    acc = lax.dot_general(x, w, (((1,), (1,)), ((), ())),
                          preferred_element_type=jnp.float32)
    n_ref[:] = jnp.tanh(_ALPHA * (acc + b_ref[:]))


def _adj_body(nrow_ref, nfull_ref, nz0_ref, nz1_ref, nz2_ref, nz3_ref,
              o0_ref, o1_ref, o2_ref, o3_ref):
    nb = nrow_ref[:]
    r = nb[:, 0 * _D:1 * _D]
    i = nb[:, 1 * _D:2 * _D]
    j = nb[:, 2 * _D:3 * _D]
    k = nb[:, 3 * _D:4 * _D]
    rows = (
        jnp.concatenate([r, -i, -j, -k], axis=1),
        jnp.concatenate([i, r, -k, j], axis=1),
        jnp.concatenate([j, k, r, -i], axis=1),
        jnp.concatenate([k, -j, i, r], axis=1),
    )
    nz_refs = (nz0_ref, nz1_ref, nz2_ref, nz3_ref)
    o_refs = (o0_ref, o1_ref, o2_ref, o3_ref)
    nfull = nfull_ref[:]
    iota = lax.broadcasted_iota(jnp.int32, (1, _N), 1)
    for t in range(4):
        a = lax.dot_general(rows[t], nfull, (((1,), (1,)), ((), ())),
                            preferred_element_type=jnp.float32)
        s = jnp.maximum(jnp.tanh(_ALPHA * a), 0.0)
        v = s + nz_refs[t][:]

        def body(step, carry):
            vw, mask = carry
            m = jnp.max(vw, axis=1, keepdims=True)
            eq = vw == m
            idx = jnp.min(jnp.where(eq, iota, _N), axis=1, keepdims=True)
            sel = iota == idx
            return jnp.where(sel, -1.0, vw), jnp.where(sel, 1.0, mask)

        _, mask = lax.fori_loop(
            0, _K, body, (v, jnp.zeros(v.shape, jnp.float32)))
        o_refs[t][:] = s * mask


def kernel(idx, emb, W, b):
    x = jnp.take(emb, idx, axis=0)
    n = pl.pallas_call(
        _embed_body,
        out_shape=jax.ShapeDtypeStruct((_N, 4 * _D), jnp.float32),
    )(x, W, b.reshape(1, 4 * _D))

    nkey = jax.random.key(42)
    noise = [jax.random.uniform(jax.random.fold_in(nkey, t), (_N, _N),
                                dtype=jnp.float32) * 0.01 for t in range(4)]

    grid = (_N // _BM,)
    oshape = jax.ShapeDtypeStruct((_N, _N), jnp.float32)
    blk = pl.BlockSpec((_BM, _N), lambda m: (m, 0))
    outs = pl.pallas_call(
        _adj_body,
        grid=grid,
        in_specs=[
            pl.BlockSpec((_BM, 4 * _D), lambda m: (m, 0)),
            pl.BlockSpec((_N, 4 * _D), lambda m: (0, 0)),
            blk, blk, blk, blk,
        ],
        out_specs=(blk, blk, blk, blk),
        out_shape=(oshape, oshape, oshape, oshape),
    )(n, n, *noise)
    return tuple(outs)


# speedup vs baseline: 5.6191x; 1.9249x over previous
import jax
import jax.numpy as jnp
from jax import lax
from jax.experimental import pallas as pl

_N = 2000
_K = 30
_D = 64
_ALPHA = 3.0
_BM = 200


def _embed_body(x_ref, w_ref, b_ref, n_ref):
    x = x_ref[:]
    w = w_ref[:]
    acc = lax.dot_general(x, w, (((1,), (1,)), ((), ())),
                          preferred_element_type=jnp.float32)
    n_ref[:] = jnp.tanh(_ALPHA * (acc + b_ref[:]))


def _adj_body(nrow_ref, nfull_ref, nz0_ref, nz1_ref, nz2_ref, nz3_ref,
              o0_ref, o1_ref, o2_ref, o3_ref):
    nb = nrow_ref[:]
    r = nb[:, 0 * _D:1 * _D]
    i = nb[:, 1 * _D:2 * _D]
    j = nb[:, 2 * _D:3 * _D]
    k = nb[:, 3 * _D:4 * _D]
    rows = (
        jnp.concatenate([r, -i, -j, -k], axis=1),
        jnp.concatenate([i, r, -k, j], axis=1),
        jnp.concatenate([j, k, r, -i], axis=1),
        jnp.concatenate([k, -j, i, r], axis=1),
    )
    nz_refs = (nz0_ref, nz1_ref, nz2_ref, nz3_ref)
    o_refs = (o0_ref, o1_ref, o2_ref, o3_ref)
    nfull = nfull_ref[:]
    for t in range(4):
        a = lax.dot_general(rows[t], nfull, (((1,), (1,)), ((), ())),
                            preferred_element_type=jnp.float32)
        s = jnp.maximum(jnp.tanh(_ALPHA * a), 0.0)
        v = s + nz_refs[t][:]
        vb = lax.bitcast_convert_type(v, jnp.int32)

        def bisect(step, carry):
            lo, hi = carry
            mid = lo + ((hi - lo + 1) >> 1)
            cnt = jnp.sum(jnp.where(vb >= mid, 1.0, 0.0), axis=1,
                          keepdims=True)
            ge = cnt >= _K
            return jnp.where(ge, mid, lo), jnp.where(ge, hi, mid - 1)

        lo0 = jnp.zeros((v.shape[0], 1), jnp.int32)
        hi0 = jnp.full((v.shape[0], 1), 0x3F828F5E, jnp.int32)
        kth, _ = lax.fori_loop(0, 30, bisect, (lo0, hi0))

        gt = vb > kth
        eq = vb == kth
        need = _K - jnp.sum(jnp.where(gt, 1.0, 0.0), axis=1, keepdims=True)
        iota = lax.broadcasted_iota(jnp.int32, v.shape, 1)

        def ibisect(step, carry):
            lo, hi = carry
            mid = (lo + hi) >> 1
            cnt = jnp.sum(jnp.where(eq & (iota <= mid), 1.0, 0.0), axis=1,
                          keepdims=True)
            ok = cnt >= need
            return jnp.where(ok, lo, mid), jnp.where(ok, mid, hi)

        lo0 = jnp.full((v.shape[0], 1), -1, jnp.int32)
        hi0 = jnp.full((v.shape[0], 1), _N - 1, jnp.int32)
        _, jsel = lax.fori_loop(0, 11, ibisect, (lo0, hi0))
        tie_ok = (iota <= jsel) & (need >= 1.0)
        mask = jnp.where(gt | (eq & tie_ok), 1.0, 0.0)
        o_refs[t][:] = s * mask


def kernel(idx, emb, W, b):
    x = jnp.take(emb, idx, axis=0)
    n = pl.pallas_call(
        _embed_body,
        out_shape=jax.ShapeDtypeStruct((_N, 4 * _D), jnp.float32),
    )(x, W, b.reshape(1, 4 * _D))

    nkey = jax.random.key(42)
    noise = [jax.random.uniform(jax.random.fold_in(nkey, t), (_N, _N),
                                dtype=jnp.float32) * 0.01 for t in range(4)]

    grid = (_N // _BM,)
    oshape = jax.ShapeDtypeStruct((_N, _N), jnp.float32)
    blk = pl.BlockSpec((_BM, _N), lambda m: (m, 0))
    outs = pl.pallas_call(
        _adj_body,
        grid=grid,
        in_specs=[
            pl.BlockSpec((_BM, 4 * _D), lambda m: (m, 0)),
            pl.BlockSpec((_N, 4 * _D), lambda m: (0, 0)),
            blk, blk, blk, blk,
        ],
        out_specs=(blk, blk, blk, blk),
        out_shape=(oshape, oshape, oshape, oshape),
    )(n, n, *noise)
    return tuple(outs)
